# baseline (speedup 1.0000x reference)
import numpy as np
import jax
import jax.numpy as jnp
from jax.experimental import pallas as pl
from jax.experimental.pallas import tpu as pltpu

NQ = 8
DEPTH = 3
PI = 3.14159
BB = 128


def _dot(a, b):
    return jnp.dot(a, b, preferred_element_type=jnp.float32)


def _qcnn_body(xp, w2, w3, gs, b1, b3, f1b, th, pq, cm, sg, zpm,
               q2ct, q2cb, bqt, bqb, bft, bfb, ia, ib, inb, f2t, f2b, o):
    relu = jax.nn.relu
    xs = xp[...]
    lhs = [xs[kh:kh + 28].reshape(28 * BB, 90) for kh in range(3)]

    def conv1(c):
        y = (_dot(lhs[0], w2[3 * c + 0]) + _dot(lhs[1], w2[3 * c + 1])
             + _dot(lhs[2], w2[3 * c + 2]))
        y4 = y.reshape(14, 2, BB, 448)
        return jnp.maximum(y4[:, 0], y4[:, 1])

    p1 = jnp.maximum(conv1(0), conv1(1))
    p1 = relu(p1 + b1[...]).reshape(7, 2, BB, 448)
    pe = p1[:, 0]
    po = p1[:, 1]

    a_slabs = {
        (0, 0): pe[0:6], (0, 1): po[0:6], (0, 2): pe[1:7],
        (1, 0): po[0:6], (1, 1): pe[1:7], (1, 2): po[1:7],
    }
    m2 = None
    for rp in (0, 1):
        a_flat = [a_slabs[(rp, kh)].reshape(6 * BB, 448) for kh in range(3)]
        for c in (0, 1):
            z = (_dot(a_flat[0], w3[2 * 0 + c]) + _dot(a_flat[1], w3[2 * 1 + c])
                 + _dot(a_flat[2], w3[2 * 2 + c]))
            m2 = z if m2 is None else jnp.maximum(m2, z)
    p2 = relu(m2 + b3[...]).reshape(6, BB, 384)

    acc = _dot(p2[0], gs[0])
    for t in range(1, 6):
        acc = acc + _dot(p2[t], gs[t])
    classical = relu(acc[:, :512] + f1b[...])
    fractal = jnp.sin(acc[:, 512:576] * PI)
    rep = acc[:, 576:584]
    nrm = jnp.sqrt(jnp.sum(rep * rep, axis=1, keepdims=True))
    qin = rep / (nrm + 1e-8)

    col = jax.lax.broadcasted_iota(jnp.int32, (BB, 256), 1)
    state = (col == 0).astype(jnp.float32)
    for d in range(DEPTH):
        ang = 0.5 * (qin + th[d:d + 1, :])
        cth = jnp.cos(ang)
        sth = jnp.sin(ang)
        for q in range(NQ):
            sw = _dot(state, pq[q])
            state = cth[:, q:q + 1] * state + sth[:, q:q + 1] * (sg[q:q + 1, :] * sw)
        state = _dot(state, cm[...])
    qout = _dot(state * state, zpm[...])

    qfeat = _dot(qout, q2ct[...]) + q2cb[...]
    qf = jnp.tanh(_dot(qfeat, bqt[...]) + bqb[...]) * \
        jnp.tanh(_dot(fractal, bft[...]) + bfb[...])
    integrated = _dot(classical, ia[...]) + _dot(qf, ib[...]) + inb[...]
    o[...] = _dot(integrated, f2t[...]) + f2b[...]


def kernel(x, conv1_w, conv1_b, conv2_w, conv2_b, fc1_w, fc1_b, fc2_w, fc2_b,
           q2c_w, q2c_b, bq_w, bq_b, bf_w, bf_b, int_w, int_b, theta):
    f32 = jnp.float32
    B = x.shape[0]
    nb = B // BB

    xpad = jnp.pad(x, ((0, 0), (0, 0), (1, 1), (1, 1)))
    xp = xpad.transpose(2, 0, 1, 3).reshape(30, B, 90)

    iw1 = np.arange(30)
    w2_list = []
    for c in range(2):
        ow = 2 * np.arange(14) + c
        kw = iw1[:, None] - ow[None, :]
        msk = jnp.asarray(((kw >= 0) & (kw <= 2)).astype(np.float32))
        kwc = np.clip(kw, 0, 2)
        g = conv1_w[:, :, :, kwc] * msk[None, None, None]
        g = g.transpose(2, 1, 3, 0, 4).reshape(3, 90, 448)
        w2_list.append(g)
    w2s = jnp.concatenate(w2_list, axis=0)

    iw2 = np.arange(14)
    w3_list = []
    for kh in range(3):
        for c in range(2):
            ow = 2 * np.arange(6) + c
            kw = iw2[:, None] - ow[None, :]
            msk = jnp.asarray(((kw >= 0) & (kw <= 2)).astype(np.float32))
            kwc = np.clip(kw, 0, 2)
            g = conv2_w[:, :, kh, kwc] * msk[None, None]
            w3_list.append(g.transpose(1, 2, 0, 3).reshape(448, 384))
    w3s = jnp.stack(w3_list)

    gfc = fc1_w.reshape(512, 64, 6, 6).transpose(2, 1, 3, 0)
    gfc = gfc.reshape(6, 384, 512)
    gmean = np.kron(np.eye(64, dtype=np.float32),
                    np.ones((6, 1), dtype=np.float32)) / 36.0
    gmean = jnp.asarray(np.broadcast_to(gmean, (6, 384, 64)))
    grep = np.zeros((6, 384, 8), dtype=np.float32)
    for k in range(8):
        grep[k // 6, (k % 6), k] = 1.0
    gs = jnp.concatenate([gfc, gmean, jnp.asarray(grep)], axis=2)

    idx = np.arange(256)
    pq = np.zeros((8, 256, 256), dtype=np.float32)
    for q in range(8):
        v = 1 << (7 - q)
        pq[q, idx ^ v, idx] = 1.0
    sg = np.where((idx[None, :] >> (7 - np.arange(8)[:, None])) & 1,
                  1.0, -1.0).astype(np.float32)
    t = idx.copy()
    for q in range(6, -1, -1):
        cv, tv = 1 << (7 - q), 1 << (6 - q)
        t = np.where(t & cv, t ^ tv, t)
    cmat = np.zeros((256, 256), dtype=np.float32)
    cmat[t, idx] = 1.0
    zpm = np.where((idx[:, None] >> (7 - np.arange(8)[None, :])) & 1,
                   -1.0, 1.0).astype(np.float32)

    b1rep = jnp.repeat(conv1_b, 14)[None]
    b3rep = jnp.repeat(conv2_b, 6)[None]

    const2 = lambda i: (0, 0)
    const3 = lambda i: (0, 0, 0)
    in_specs = [
        pl.BlockSpec((30, BB, 90), lambda i: (0, i, 0)),
        pl.BlockSpec((6, 90, 448), const3),
        pl.BlockSpec((6, 448, 384), const3),
        pl.BlockSpec((6, 384, 584), const3),
        pl.BlockSpec((1, 448), const2),
        pl.BlockSpec((1, 384), const2),
        pl.BlockSpec((1, 512), const2),
        pl.BlockSpec((3, 8), const2),
        pl.BlockSpec((8, 256, 256), const3),
        pl.BlockSpec((256, 256), const2),
        pl.BlockSpec((8, 256), const2),
        pl.BlockSpec((256, 8), const2),
        pl.BlockSpec((8, 64), const2),
        pl.BlockSpec((1, 64), const2),
        pl.BlockSpec((64, 32), const2),
        pl.BlockSpec((1, 32), const2),
        pl.BlockSpec((64, 32), const2),
        pl.BlockSpec((1, 32), const2),
        pl.BlockSpec((512, 512), const2),
        pl.BlockSpec((32, 512), const2),
        pl.BlockSpec((1, 512), const2),
        pl.BlockSpec((512, 10), const2),
        pl.BlockSpec((1, 10), const2),
    ]
    out = pl.pallas_call(
        _qcnn_body,
        grid=(nb,),
        in_specs=in_specs,
        out_specs=pl.BlockSpec((BB, 10), lambda i: (i, 0)),
        out_shape=jax.ShapeDtypeStruct((B, 10), f32),
        compiler_params=pltpu.CompilerParams(
            dimension_semantics=("parallel",),
            vmem_limit_bytes=56 * 1024 * 1024,
        ),
        name="qcnn_fused",
    )(xp, w2s, w3s, gs, b1rep, b3rep, fc1_b[None],
      theta, jnp.asarray(pq), jnp.asarray(cmat), jnp.asarray(sg),
      jnp.asarray(zpm), q2c_w.T, q2c_b[None], bq_w.T, bq_b[None], bf_w.T,
      bf_b[None], int_w[:, :512].T, int_w[:, 512:].T, int_b[None],
      fc2_w.T, fc2_b[None])
    return out


# speedup vs baseline: 1.1557x; 1.1557x over previous
import numpy as np
import jax
import jax.numpy as jnp
from jax.experimental import pallas as pl
from jax.experimental.pallas import tpu as pltpu

NQ = 8
DEPTH = 3
PI = 3.14159
BB = 128


def _dot(a, b):
    return jnp.dot(a, b, preferred_element_type=jnp.float32)


def _dott(a, b):
    return jax.lax.dot_general(a, b, (((1,), (1,)), ((), ())),
                               preferred_element_type=jnp.float32)


def _qcnn_body(xp, w2, w3, gs, bp, th, pq, cm, sg, zpm,
               q2c, q2cb, bq, bqb, bf, bfb, iw, inb, f2, f2b, o):
    relu = jax.nn.relu
    xs = xp[...]
    lhs = [xs[kh:kh + 28].reshape(28 * BB, 90) for kh in range(3)]

    def conv1(c):
        y = (_dot(lhs[0], w2[3 * c + 0]) + _dot(lhs[1], w2[3 * c + 1])
             + _dot(lhs[2], w2[3 * c + 2]))
        y4 = y.reshape(14, 2, BB, 448)
        return jnp.maximum(y4[:, 0], y4[:, 1])

    p1 = jnp.maximum(conv1(0), conv1(1))
    p1 = relu(p1 + bp[:, :448]).reshape(7, 2, BB, 448)
    pe = p1[:, 0]
    po = p1[:, 1]

    a_slabs = {
        (0, 0): pe[0:6], (0, 1): po[0:6], (0, 2): pe[1:7],
        (1, 0): po[0:6], (1, 1): pe[1:7], (1, 2): po[1:7],
    }
    m2 = None
    for rp in (0, 1):
        a_flat = [a_slabs[(rp, kh)].reshape(6 * BB, 448) for kh in range(3)]
        for c in (0, 1):
            z = (_dot(a_flat[0], w3[2 * 0 + c]) + _dot(a_flat[1], w3[2 * 1 + c])
                 + _dot(a_flat[2], w3[2 * 2 + c]))
            m2 = z if m2 is None else jnp.maximum(m2, z)
    p2 = relu(m2 + bp[:, 448:832]).reshape(6, BB, 384)

    acc = _dot(p2[0], gs[0])
    for t in range(1, 6):
        acc = acc + _dot(p2[t], gs[t])
    classical = relu(acc[:, :512] + bp[:, 832:1344])
    fractal = jnp.sin(acc[:, 512:576] * PI)
    rep = acc[:, 576:584]
    nrm = jnp.sqrt(jnp.sum(rep * rep, axis=1, keepdims=True))
    qin = rep / (nrm + 1e-8)

    col = jax.lax.broadcasted_iota(jnp.int32, (BB, 256), 1)
    state = (col == 0).astype(jnp.float32)
    for d in range(DEPTH):
        ang = 0.5 * (qin + th[d:d + 1, :])
        cth = jnp.cos(ang)
        sth = jnp.sin(ang)
        for q in range(NQ):
            sw = _dot(state, pq[q])
            state = cth[:, q:q + 1] * state + sth[:, q:q + 1] * (sg[q:q + 1, :] * sw)
        state = _dot(state, cm[...])
    qout = _dot(state * state, zpm[...])

    qfeat = _dott(qout, q2c[...]) + q2cb[...]
    qf = jnp.tanh(_dott(qfeat, bq[...]) + bqb[...]) * \
        jnp.tanh(_dott(fractal, bf[...]) + bfb[...])
    iwv = iw[...]
    integrated = _dott(classical, iwv[:, :512]) + _dott(qf, iwv[:, 512:544]) \
        + inb[...]
    o[...] = _dott(integrated, f2[...]) + f2b[...]


def kernel(x, conv1_w, conv1_b, conv2_w, conv2_b, fc1_w, fc1_b, fc2_w, fc2_b,
           q2c_w, q2c_b, bq_w, bq_b, bf_w, bf_b, int_w, int_b, theta):
    f32 = jnp.float32
    B = x.shape[0]
    nb = B // BB

    xpad = jnp.pad(x, ((0, 0), (0, 0), (1, 1), (1, 1)))
    xp = xpad.transpose(2, 0, 1, 3).reshape(30, B, 90)

    kwv = np.arange(3)
    s2 = ((np.arange(30)[None, :, None, None] - kwv[:, None, None, None]
           - 2 * np.arange(14)[None, None, :, None]
           - np.arange(2)[None, None, None, :]) == 0).astype(np.float32)
    w2s = jnp.einsum('oihk,kpuc->chipou', conv1_w,
                     jnp.asarray(s2)).reshape(6, 90, 448)

    s3 = ((np.arange(14)[None, :, None, None] - kwv[:, None, None, None]
           - 2 * np.arange(6)[None, None, :, None]
           - np.arange(2)[None, None, None, :]) == 0).astype(np.float32)
    w3s = jnp.einsum('oihk,kpuc->hcipou', conv2_w,
                     jnp.asarray(s3)).reshape(6, 448, 384)

    gfc = fc1_w.reshape(512, 64, 6, 6).transpose(2, 1, 3, 0)
    gfc = gfc.reshape(6, 384, 512)
    gmean = np.kron(np.eye(64, dtype=np.float32),
                    np.ones((6, 1), dtype=np.float32)) / 36.0
    gmean = np.broadcast_to(gmean, (6, 384, 64))
    grep = np.zeros((6, 384, 8), dtype=np.float32)
    for k in range(8):
        grep[k // 6, (k % 6), k] = 1.0
    gaux = jnp.asarray(np.concatenate([gmean, grep], axis=2))
    gs = jnp.concatenate([gfc, gaux], axis=2)

    idx = np.arange(256)
    pq = np.zeros((8, 256, 256), dtype=np.float32)
    for q in range(8):
        v = 1 << (7 - q)
        pq[q, idx ^ v, idx] = 1.0
    sg = np.where((idx[None, :] >> (7 - np.arange(8)[:, None])) & 1,
                  1.0, -1.0).astype(np.float32)
    t = idx.copy()
    for q in range(6, -1, -1):
        cv, tv = 1 << (7 - q), 1 << (6 - q)
        t = np.where(t & cv, t ^ tv, t)
    cmat = np.zeros((256, 256), dtype=np.float32)
    cmat[t, idx] = 1.0
    zpm = np.where((idx[:, None] >> (7 - np.arange(8)[None, :])) & 1,
                   -1.0, 1.0).astype(np.float32)

    bias_pack = jnp.concatenate([jnp.repeat(conv1_b, 14),
                                 jnp.repeat(conv2_b, 6), fc1_b])[None]

    const2 = lambda i: (0, 0)
    const3 = lambda i: (0, 0, 0)
    in_specs = [
        pl.BlockSpec((30, BB, 90), lambda i: (0, i, 0)),
        pl.BlockSpec((6, 90, 448), const3),
        pl.BlockSpec((6, 448, 384), const3),
        pl.BlockSpec((6, 384, 584), const3),
        pl.BlockSpec((1, 1344), const2),
        pl.BlockSpec((3, 8), const2),
        pl.BlockSpec((8, 256, 256), const3),
        pl.BlockSpec((256, 256), const2),
        pl.BlockSpec((8, 256), const2),
        pl.BlockSpec((256, 8), const2),
        pl.BlockSpec((64, 8), const2),
        pl.BlockSpec((1, 64), const2),
        pl.BlockSpec((32, 64), const2),
        pl.BlockSpec((1, 32), const2),
        pl.BlockSpec((32, 64), const2),
        pl.BlockSpec((1, 32), const2),
        pl.BlockSpec((512, 544), const2),
        pl.BlockSpec((1, 512), const2),
        pl.BlockSpec((10, 512), const2),
        pl.BlockSpec((1, 10), const2),
    ]
    out = pl.pallas_call(
        _qcnn_body,
        grid=(nb,),
        in_specs=in_specs,
        out_specs=pl.BlockSpec((BB, 10), lambda i: (i, 0)),
        out_shape=jax.ShapeDtypeStruct((B, 10), f32),
        compiler_params=pltpu.CompilerParams(
            dimension_semantics=("parallel",),
            vmem_limit_bytes=56 * 1024 * 1024,
        ),
        name="qcnn_fused",
    )(xp, w2s, w3s, gs, bias_pack, theta, jnp.asarray(pq), jnp.asarray(cmat),
      jnp.asarray(sg), jnp.asarray(zpm), q2c_w, q2c_b[None], bq_w, bq_b[None],
      bf_w, bf_b[None], int_w, int_b[None], fc2_w, fc2_b[None])
    return out


# speedup vs baseline: 1.1962x; 1.0350x over previous
import numpy as np
import jax
import jax.numpy as jnp
from jax.experimental import pallas as pl
from jax.experimental.pallas import tpu as pltpu

NQ = 8
DEPTH = 3
PI = 3.14159
BB = 128


def _dot(a, b):
    return jnp.dot(a, b, preferred_element_type=jnp.float32)


def _dott(a, b):
    return jax.lax.dot_general(a, b, (((1,), (1,)), ((), ())),
                               preferred_element_type=jnp.float32)


def _qcnn_body(xp, w2, w3, gs, bp, th, pq, cm, sg, zpm,
               q2c, q2cb, bq, bqb, bf, bfb, iw, inb, f2, f2b, o):
    relu = jax.nn.relu
    xs = xp[...]
    lhs = [xs[kh:kh + 28].reshape(28 * BB, 90) for kh in range(3)]

    def conv1(c):
        y = (_dot(lhs[0], w2[3 * c + 0]) + _dot(lhs[1], w2[3 * c + 1])
             + _dot(lhs[2], w2[3 * c + 2]))
        y4 = y.reshape(14, 2, BB, 448)
        return jnp.maximum(y4[:, 0], y4[:, 1])

    p1 = jnp.maximum(conv1(0), conv1(1))
    p1 = relu(p1 + bp[:, :448]).reshape(7, 2, BB, 448)
    pe = p1[:, 0]
    po = p1[:, 1]

    a_slabs = {
        (0, 0): pe[0:6], (0, 1): po[0:6], (0, 2): pe[1:7],
        (1, 0): po[0:6], (1, 1): pe[1:7], (1, 2): po[1:7],
    }
    m2 = None
    for rp in (0, 1):
        a_flat = [a_slabs[(rp, kh)].reshape(6 * BB, 448) for kh in range(3)]
        for c in (0, 1):
            z = (_dot(a_flat[0], w3[2 * 0 + c]) + _dot(a_flat[1], w3[2 * 1 + c])
                 + _dot(a_flat[2], w3[2 * 2 + c]))
            m2 = z if m2 is None else jnp.maximum(m2, z)
    p2 = relu(m2 + bp[:, 448:832]).reshape(6, BB, 384)

    acc = _dot(p2[0], gs[0])
    for t in range(1, 6):
        acc = acc + _dot(p2[t], gs[t])
    classical = relu(acc[:, :512] + bp[:, 832:1344])
    fractal = jnp.sin(acc[:, 512:576] * PI)
    rep = acc[:, 576:584]
    nrm = jnp.sqrt(jnp.sum(rep * rep, axis=1, keepdims=True))
    qin = rep / (nrm + 1e-8)

    col = jax.lax.broadcasted_iota(jnp.int32, (BB, 256), 1)
    state = (col == 0).astype(jnp.float32)
    for d in range(DEPTH):
        ang = 0.5 * (qin + th[d:d + 1, :])
        cth = jnp.cos(ang)
        sth = jnp.sin(ang)
        for q in range(NQ):
            sw = _dot(state, pq[q])
            state = cth[:, q:q + 1] * state + sth[:, q:q + 1] * (sg[q:q + 1, :] * sw)
        state = _dot(state, cm[...])
    qout = _dot(state * state, zpm[...])

    qfeat = _dott(qout, q2c[...]) + q2cb[...]
    qf = jnp.tanh(_dott(qfeat, bq[...]) + bqb[...]) * \
        jnp.tanh(_dott(fractal, bf[...]) + bfb[...])
    iwv = iw[...]
    integrated = _dott(classical, iwv[:, :512]) + _dott(qf, iwv[:, 512:544]) \
        + inb[...]
    o[...] = _dott(integrated, f2[...]) + f2b[...]


def kernel(x, conv1_w, conv1_b, conv2_w, conv2_b, fc1_w, fc1_b, fc2_w, fc2_b,
           q2c_w, q2c_b, bq_w, bq_b, bf_w, bf_b, int_w, int_b, theta):
    f32 = jnp.float32
    B = x.shape[0]
    nb = B // BB

    xpad = jnp.pad(x, ((0, 0), (0, 0), (1, 1), (1, 1)))
    xp = xpad.transpose(2, 0, 1, 3).reshape(30, B, 90)

    kwv = np.arange(3)
    s2 = ((np.arange(30)[None, :, None, None] - kwv[:, None, None, None]
           - 2 * np.arange(14)[None, None, :, None]
           - np.arange(2)[None, None, None, :]) == 0).astype(np.float32)
    w2s = jnp.einsum('oihk,kpuc->chipou', conv1_w,
                     jnp.asarray(s2)).reshape(6, 90, 448)

    s3 = ((np.arange(14)[None, :, None, None] - kwv[:, None, None, None]
           - 2 * np.arange(6)[None, None, :, None]
           - np.arange(2)[None, None, None, :]) == 0).astype(np.float32)
    w3s = jnp.einsum('oihk,kpuc->hcipou', conv2_w,
                     jnp.asarray(s3)).reshape(6, 448, 384)

    gfc = fc1_w.reshape(512, 64, 6, 6).transpose(2, 1, 3, 0)
    gfc = gfc.reshape(6, 384, 512)
    gmean = np.kron(np.eye(64, dtype=np.float32),
                    np.ones((6, 1), dtype=np.float32)) / 36.0
    gmean = np.broadcast_to(gmean, (6, 384, 64))
    grep = np.zeros((6, 384, 8), dtype=np.float32)
    for k in range(8):
        grep[k // 6, (k % 6), k] = 1.0
    gaux = jnp.asarray(np.concatenate([gmean, grep], axis=2))
    gs = jnp.concatenate([gfc, gaux], axis=2)

    idx = np.arange(256)
    pq = np.zeros((8, 256, 256), dtype=np.float32)
    for q in range(8):
        v = 1 << (7 - q)
        pq[q, idx ^ v, idx] = 1.0
    sg = np.where((idx[None, :] >> (7 - np.arange(8)[:, None])) & 1,
                  1.0, -1.0).astype(np.float32)
    t = idx.copy()
    for q in range(6, -1, -1):
        cv, tv = 1 << (7 - q), 1 << (6 - q)
        t = np.where(t & cv, t ^ tv, t)
    cmat = np.zeros((256, 256), dtype=np.float32)
    cmat[t, idx] = 1.0
    zpm = np.where((idx[:, None] >> (7 - np.arange(8)[None, :])) & 1,
                   -1.0, 1.0).astype(np.float32)

    bias_pack = jnp.concatenate([jnp.repeat(conv1_b, 14),
                                 jnp.repeat(conv2_b, 6), fc1_b])[None]

    const2 = lambda i: (0, 0)
    const3 = lambda i: (0, 0, 0)
    in_specs = [
        pl.BlockSpec((30, BB, 90), lambda i: (0, i, 0)),
        pl.BlockSpec((6, 90, 448), const3),
        pl.BlockSpec((6, 448, 384), const3),
        pl.BlockSpec((6, 384, 584), const3),
        pl.BlockSpec((1, 1344), const2),
        pl.BlockSpec((3, 8), const2),
        pl.BlockSpec((8, 256, 256), const3),
        pl.BlockSpec((256, 256), const2),
        pl.BlockSpec((8, 256), const2),
        pl.BlockSpec((256, 8), const2),
        pl.BlockSpec((64, 8), const2),
        pl.BlockSpec((1, 64), const2),
        pl.BlockSpec((32, 64), const2),
        pl.BlockSpec((1, 32), const2),
        pl.BlockSpec((32, 64), const2),
        pl.BlockSpec((1, 32), const2),
        pl.BlockSpec((512, 544), const2),
        pl.BlockSpec((1, 512), const2),
        pl.BlockSpec((10, 512), const2),
        pl.BlockSpec((1, 10), const2),
    ]
    out = pl.pallas_call(
        _qcnn_body,
        grid=(nb,),
        in_specs=in_specs,
        out_specs=pl.BlockSpec((BB, 10), lambda i: (i, 0)),
        out_shape=jax.ShapeDtypeStruct((B, 10), f32),
        compiler_params=pltpu.CompilerParams(
            dimension_semantics=("parallel",),
            allow_input_fusion=(True,) * 20,
            vmem_limit_bytes=56 * 1024 * 1024,
        ),
        name="qcnn_fused",
    )(xp, w2s, w3s, gs, bias_pack, theta, jnp.asarray(pq), jnp.asarray(cmat),
      jnp.asarray(sg), jnp.asarray(zpm), q2c_w, q2c_b[None], bq_w, bq_b[None],
      bf_w, bf_b[None], int_w, int_b[None], fc2_w, fc2_b[None])
    return out


# speedup vs baseline: 1.2966x; 1.0840x over previous
import numpy as np
import jax
import jax.numpy as jnp
from jax.experimental import pallas as pl
from jax.experimental.pallas import tpu as pltpu

NQ = 8
DEPTH = 3
PI = 3.14159
BB = 128


def _dot(a, b):
    return jnp.dot(a, b, preferred_element_type=jnp.float32)


def _dott(a, b):
    return jax.lax.dot_general(a, b, (((1,), (1,)), ((), ())),
                               preferred_element_type=jnp.float32)


def _qcnn_body(xp, w2, w3, gs, bp, th, qm, zpm,
               q2c, q2cb, bq, bqb, bf, bfb, iw, inb, f2, f2b, o):
    relu = jax.nn.relu
    xs = xp[...]
    lhs = jnp.concatenate([xs[0:28], xs[1:29], xs[2:30]],
                          axis=2).reshape(28 * BB, 270)

    def conv1(c):
        y4 = _dot(lhs, w2[c]).reshape(14, 2, BB, 448)
        return jnp.maximum(y4[:, 0], y4[:, 1])

    p1 = jnp.maximum(conv1(0), conv1(1))
    p1 = relu(p1 + bp[:, :448]).reshape(7, 2, BB, 448)
    pe = p1[:, 0]
    po = p1[:, 1]

    a_slabs = {
        (0, 0): pe[0:6], (0, 1): po[0:6], (0, 2): pe[1:7],
        (1, 0): po[0:6], (1, 1): pe[1:7], (1, 2): po[1:7],
    }
    m2 = None
    for rp in (0, 1):
        a_flat = [a_slabs[(rp, kh)].reshape(6 * BB, 448) for kh in range(3)]
        z = (_dot(a_flat[0], w3[0]) + _dot(a_flat[1], w3[1])
             + _dot(a_flat[2], w3[2]))
        zz = jnp.maximum(z[:, :384], z[:, 384:])
        m2 = zz if m2 is None else jnp.maximum(m2, zz)
    p2 = relu(m2 + bp[:, 448:832]).reshape(6, BB, 384)

    acc = _dot(p2[0], gs[0])
    for t in range(1, 6):
        acc = acc + _dot(p2[t], gs[t])
    classical = relu(acc[:, :512] + bp[:, 832:1344])
    fractal = jnp.sin(acc[:, 512:576] * PI)
    rep = acc[:, 576:584]
    nrm = jnp.sqrt(jnp.sum(rep * rep, axis=1, keepdims=True))
    qin = rep / (nrm + 1e-8)

    col = jax.lax.broadcasted_iota(jnp.int32, (BB, 256), 1)
    state = (col == 0).astype(jnp.float32)
    for d in range(DEPTH):
        ang = 0.5 * (qin + th[d:d + 1, :])
        cth = jnp.cos(ang)
        sth = jnp.sin(ang)
        for j in range(4):
            qa, qb = 2 * j, 2 * j + 1
            ca, sa = cth[:, qa:qa + 1], sth[:, qa:qa + 1]
            cb, sb = cth[:, qb:qb + 1], sth[:, qb:qb + 1]
            if j < 3:
                s0 = state
                sA = _dot(state, qm[3 * j + 0])
                sB = _dot(state, qm[3 * j + 1])
                sC = _dot(state, qm[3 * j + 2])
            else:
                s0 = _dot(state, qm[9])
                sA = _dot(state, qm[10])
                sB = _dot(state, qm[11])
                sC = _dot(state, qm[12])
            state = ((ca * cb) * s0 + (sa * cb) * sA
                     + (ca * sb) * sB + (sa * sb) * sC)
    qout = _dot(state * state, zpm[...])

    qfeat = _dott(qout, q2c[...]) + q2cb[...]
    qf = jnp.tanh(_dott(qfeat, bq[...]) + bqb[...]) * \
        jnp.tanh(_dott(fractal, bf[...]) + bfb[...])
    iwv = iw[...]
    integrated = _dott(classical, iwv[:, :512]) + _dott(qf, iwv[:, 512:544]) \
        + inb[...]
    o[...] = _dott(integrated, f2[...]) + f2b[...]


def kernel(x, conv1_w, conv1_b, conv2_w, conv2_b, fc1_w, fc1_b, fc2_w, fc2_b,
           q2c_w, q2c_b, bq_w, bq_b, bf_w, bf_b, int_w, int_b, theta):
    f32 = jnp.float32
    B = x.shape[0]
    nb = B // BB

    xpad = jnp.pad(x, ((0, 0), (0, 0), (1, 1), (1, 1)))
    xp = xpad.transpose(2, 0, 1, 3).reshape(30, B, 90)

    kwv = np.arange(3)
    s2 = ((np.arange(30)[None, :, None, None] - kwv[:, None, None, None]
           - 2 * np.arange(14)[None, None, :, None]
           - np.arange(2)[None, None, None, :]) == 0).astype(np.float32)
    w2s = jnp.einsum('oihk,kpuc->chipou', conv1_w,
                     jnp.asarray(s2)).reshape(2, 270, 448)

    s3 = ((np.arange(14)[None, :, None, None] - kwv[:, None, None, None]
           - 2 * np.arange(6)[None, None, :, None]
           - np.arange(2)[None, None, None, :]) == 0).astype(np.float32)
    w3s = jnp.einsum('oihk,kpuc->hipcou', conv2_w,
                     jnp.asarray(s3)).reshape(3, 448, 768)

    gfc = fc1_w.reshape(512, 64, 6, 6).transpose(2, 1, 3, 0)
    gfc = gfc.reshape(6, 384, 512)
    gmean = np.kron(np.eye(64, dtype=np.float32),
                    np.ones((6, 1), dtype=np.float32)) / 36.0
    gmean = np.broadcast_to(gmean, (6, 384, 64))
    grep = np.zeros((6, 384, 8), dtype=np.float32)
    for k in range(8):
        grep[k // 6, (k % 6), k] = 1.0
    gaux = jnp.asarray(np.concatenate([gmean, grep], axis=2))
    gs = jnp.concatenate([gfc, gaux], axis=2)

    idx = np.arange(256)
    pq = np.zeros((8, 256, 256), dtype=np.float32)
    for q in range(8):
        v = 1 << (7 - q)
        pq[q, idx ^ v, idx] = 1.0
    sg = np.where((idx[None, :] >> (7 - np.arange(8)[:, None])) & 1,
                  1.0, -1.0).astype(np.float32)
    t = idx.copy()
    for q in range(6, -1, -1):
        cv, tv = 1 << (7 - q), 1 << (6 - q)
        t = np.where(t & cv, t ^ tv, t)
    cmat = np.zeros((256, 256), dtype=np.float32)
    cmat[t, idx] = 1.0
    zpm = np.where((idx[:, None] >> (7 - np.arange(8)[None, :])) & 1,
                   -1.0, 1.0).astype(np.float32)
    qmat = np.zeros((13, 256, 256), dtype=np.float32)
    for j in range(4):
        a = pq[2 * j] * sg[2 * j][None, :]
        b = pq[2 * j + 1] * sg[2 * j + 1][None, :]
        c = a @ b
        if j < 3:
            qmat[3 * j + 0], qmat[3 * j + 1], qmat[3 * j + 2] = a, b, c
        else:
            qmat[9] = cmat
            qmat[10] = a @ cmat
            qmat[11] = b @ cmat
            qmat[12] = c @ cmat

    bias_pack = jnp.concatenate([jnp.repeat(conv1_b, 14),
                                 jnp.repeat(conv2_b, 6), fc1_b])[None]

    const2 = lambda i: (0, 0)
    const3 = lambda i: (0, 0, 0)
    in_specs = [
        pl.BlockSpec((30, BB, 90), lambda i: (0, i, 0)),
        pl.BlockSpec((2, 270, 448), const3),
        pl.BlockSpec((3, 448, 768), const3),
        pl.BlockSpec((6, 384, 584), const3),
        pl.BlockSpec((1, 1344), const2),
        pl.BlockSpec((3, 8), const2),
        pl.BlockSpec((13, 256, 256), const3),
        pl.BlockSpec((256, 8), const2),
        pl.BlockSpec((64, 8), const2),
        pl.BlockSpec((1, 64), const2),
        pl.BlockSpec((32, 64), const2),
        pl.BlockSpec((1, 32), const2),
        pl.BlockSpec((32, 64), const2),
        pl.BlockSpec((1, 32), const2),
        pl.BlockSpec((512, 544), const2),
        pl.BlockSpec((1, 512), const2),
        pl.BlockSpec((10, 512), const2),
        pl.BlockSpec((1, 10), const2),
    ]
    out = pl.pallas_call(
        _qcnn_body,
        grid=(nb,),
        in_specs=in_specs,
        out_specs=pl.BlockSpec((BB, 10), lambda i: (i, 0)),
        out_shape=jax.ShapeDtypeStruct((B, 10), f32),
        compiler_params=pltpu.CompilerParams(
            dimension_semantics=("parallel",),
            allow_input_fusion=(True,) * 18,
            vmem_limit_bytes=56 * 1024 * 1024,
        ),
        name="qcnn_fused",
    )(xp, w2s, w3s, gs, bias_pack, theta, jnp.asarray(qmat),
      jnp.asarray(zpm), q2c_w, q2c_b[None], bq_w, bq_b[None],
      bf_w, bf_b[None], int_w, int_b[None], fc2_w, fc2_b[None])
    return out


# speedup vs baseline: 1.4349x; 1.1066x over previous
import numpy as np
import jax
import jax.numpy as jnp
from jax.experimental import pallas as pl
from jax.experimental.pallas import tpu as pltpu

NQ = 8
DEPTH = 3
PI = 3.14159
BB = 128


def _dot(a, b):
    return jnp.dot(a, b, preferred_element_type=jnp.float32)


def _dott(a, b):
    return jax.lax.dot_general(a, b, (((1,), (1,)), ((), ())),
                               preferred_element_type=jnp.float32)


def _qcnn_body(xp, w2, w3, gf, ga, bp, th, qm, zpm,
               q2c, q2cb, bq, bqb, bf, bfb, iw, inb, f2, f2b, o):
    relu = jax.nn.relu
    xs = xp[...]
    lhs = jnp.concatenate([xs[0:28], xs[1:29], xs[2:30]],
                          axis=2).reshape(28 * BB, 252)

    def conv1(c):
        y4 = _dot(lhs, w2[c]).reshape(14, 2, BB, 448)
        return jnp.maximum(y4[:, 0], y4[:, 1])

    p1 = jnp.maximum(conv1(0), conv1(1))
    p1 = relu(p1 + bp[:, :448]).reshape(7, 2, BB, 448)
    pe = p1[:, 0]
    po = p1[:, 1]

    a_slabs = {
        (0, 0): pe[0:6], (0, 1): po[0:6], (0, 2): pe[1:7],
        (1, 0): po[0:6], (1, 1): pe[1:7], (1, 2): po[1:7],
    }
    m2 = None
    for rp in (0, 1):
        a_flat = [a_slabs[(rp, kh)].reshape(6 * BB, 448) for kh in range(3)]
        z = (_dot(a_flat[0], w3[0]) + _dot(a_flat[1], w3[1])
             + _dot(a_flat[2], w3[2]))
        zz = jnp.maximum(z[:, :384], z[:, 384:])
        m2 = zz if m2 is None else jnp.maximum(m2, zz)
    p2 = relu(m2 + bp[:, 448:832]).reshape(6, BB, 384)

    acc = _dot(p2[0], gf[0])
    aux = _dot(p2[0], ga[0])
    for t in range(1, 6):
        acc = acc + _dot(p2[t], gf[t])
        aux = aux + _dot(p2[t], ga[t])
    classical = relu(acc + bp[:, 832:1344])
    fractal = jnp.sin(aux[:, :64] * PI)
    rep = aux[:, 64:72]
    nrm = jnp.sqrt(jnp.sum(rep * rep, axis=1, keepdims=True))
    qin = rep / (nrm + 1e-8)

    col = jax.lax.broadcasted_iota(jnp.int32, (BB, 256), 1)
    state = (col == 0).astype(jnp.float32)
    cq = jnp.cos(0.5 * qin)
    sq = jnp.sin(0.5 * qin)
    thv = th[...]
    ct = jnp.cos(0.5 * thv)
    st = jnp.sin(0.5 * thv)
    for d in range(DEPTH):
        cth = ct[d:d + 1, :] * cq - st[d:d + 1, :] * sq
        sth = st[d:d + 1, :] * cq + ct[d:d + 1, :] * sq
        for j in range(4):
            qa, qb = 2 * j, 2 * j + 1
            ca, sa = cth[:, qa:qa + 1], sth[:, qa:qa + 1]
            cb, sb = cth[:, qb:qb + 1], sth[:, qb:qb + 1]
            if j < 3:
                s0 = state
                sA = _dot(state, qm[3 * j + 0])
                sB = _dot(state, qm[3 * j + 1])
                sC = _dot(state, qm[3 * j + 2])
            else:
                s0 = _dot(state, qm[9])
                sA = _dot(state, qm[10])
                sB = _dot(state, qm[11])
                sC = _dot(state, qm[12])
            state = ((ca * cb) * s0 + (sa * cb) * sA
                     + (ca * sb) * sB + (sa * sb) * sC)
    qout = _dot(state * state, zpm[...])

    qfeat = _dott(qout, q2c[...]) + q2cb[...]
    qf = jnp.tanh(_dott(qfeat, bq[...]) + bqb[...]) * \
        jnp.tanh(_dott(fractal, bf[...]) + bfb[...])
    iwv = iw[...]
    integrated = _dott(classical, iwv[:, :512]) + _dott(qf, iwv[:, 512:544]) \
        + inb[...]
    o[...] = _dott(integrated, f2[...]) + f2b[...]


def kernel(x, conv1_w, conv1_b, conv2_w, conv2_b, fc1_w, fc1_b, fc2_w, fc2_b,
           q2c_w, q2c_b, bq_w, bq_b, bf_w, bf_b, int_w, int_b, theta):
    f32 = jnp.float32
    B = x.shape[0]
    nb = B // BB

    xpad = jnp.pad(x, ((0, 0), (0, 0), (1, 1), (0, 0)))
    xp = xpad.transpose(2, 0, 1, 3).reshape(30, B, 84)

    kwv = np.arange(3)
    s2 = ((np.arange(28)[None, :, None, None] + 1 - kwv[:, None, None, None]
           - 2 * np.arange(14)[None, None, :, None]
           - np.arange(2)[None, None, None, :]) == 0).astype(np.float32)
    w2s = jnp.einsum('oihk,kpuc->chipou', conv1_w,
                     jnp.asarray(s2)).reshape(2, 252, 448)

    s3 = ((np.arange(14)[None, :, None, None] - kwv[:, None, None, None]
           - 2 * np.arange(6)[None, None, :, None]
           - np.arange(2)[None, None, None, :]) == 0).astype(np.float32)
    w3s = jnp.einsum('oihk,kpuc->hipcou', conv2_w,
                     jnp.asarray(s3)).reshape(3, 448, 768)

    gfc = fc1_w.reshape(512, 64, 6, 6).transpose(2, 1, 3, 0)
    gfc = gfc.reshape(6, 384, 512)
    gmean = np.kron(np.eye(64, dtype=np.float32),
                    np.ones((6, 1), dtype=np.float32)) / 36.0
    gmean = np.broadcast_to(gmean, (6, 384, 64))
    grep = np.zeros((6, 384, 8), dtype=np.float32)
    for k in range(8):
        grep[k // 6, (k % 6), k] = 1.0
    gaux = jnp.asarray(np.concatenate([gmean, grep], axis=2))

    idx = np.arange(256)
    pq = np.zeros((8, 256, 256), dtype=np.float32)
    for q in range(8):
        v = 1 << (7 - q)
        pq[q, idx ^ v, idx] = 1.0
    sg = np.where((idx[None, :] >> (7 - np.arange(8)[:, None])) & 1,
                  1.0, -1.0).astype(np.float32)
    t = idx.copy()
    for q in range(6, -1, -1):
        cv, tv = 1 << (7 - q), 1 << (6 - q)
        t = np.where(t & cv, t ^ tv, t)
    cmat = np.zeros((256, 256), dtype=np.float32)
    cmat[t, idx] = 1.0
    zpm = np.where((idx[:, None] >> (7 - np.arange(8)[None, :])) & 1,
                   -1.0, 1.0).astype(np.float32)
    qmat = np.zeros((13, 256, 256), dtype=np.float32)
    for j in range(4):
        a = pq[2 * j] * sg[2 * j][None, :]
        b = pq[2 * j + 1] * sg[2 * j + 1][None, :]
        c = a @ b
        if j < 3:
            qmat[3 * j + 0], qmat[3 * j + 1], qmat[3 * j + 2] = a, b, c
        else:
            qmat[9] = cmat
            qmat[10] = a @ cmat
            qmat[11] = b @ cmat
            qmat[12] = c @ cmat

    bias_pack = jnp.concatenate([jnp.repeat(conv1_b, 14),
                                 jnp.repeat(conv2_b, 6), fc1_b])[None]

    const2 = lambda i: (0, 0)
    const3 = lambda i: (0, 0, 0)
    in_specs = [
        pl.BlockSpec((30, BB, 84), lambda i: (0, i, 0)),
        pl.BlockSpec((2, 252, 448), const3),
        pl.BlockSpec((3, 448, 768), const3),
        pl.BlockSpec((6, 384, 512), const3),
        pl.BlockSpec((6, 384, 72), const3),
        pl.BlockSpec((1, 1344), const2),
        pl.BlockSpec((3, 8), const2),
        pl.BlockSpec((13, 256, 256), const3),
        pl.BlockSpec((256, 8), const2),
        pl.BlockSpec((64, 8), const2),
        pl.BlockSpec((1, 64), const2),
        pl.BlockSpec((32, 64), const2),
        pl.BlockSpec((1, 32), const2),
        pl.BlockSpec((32, 64), const2),
        pl.BlockSpec((1, 32), const2),
        pl.BlockSpec((512, 544), const2),
        pl.BlockSpec((1, 512), const2),
        pl.BlockSpec((10, 512), const2),
        pl.BlockSpec((1, 10), const2),
    ]
    out = pl.pallas_call(
        _qcnn_body,
        grid=(nb,),
        in_specs=in_specs,
        out_specs=pl.BlockSpec((BB, 10), lambda i: (i, 0)),
        out_shape=jax.ShapeDtypeStruct((B, 10), f32),
        compiler_params=pltpu.CompilerParams(
            dimension_semantics=("parallel",),
            allow_input_fusion=(True,) * 19,
            vmem_limit_bytes=56 * 1024 * 1024,
        ),
        name="qcnn_fused",
    )(xp, w2s, w3s, gfc, gaux, bias_pack, theta, jnp.asarray(qmat),
      jnp.asarray(zpm), q2c_w, q2c_b[None], bq_w, bq_b[None],
      bf_w, bf_b[None], int_w, int_b[None], fc2_w, fc2_b[None])
    return out


# speedup vs baseline: 1.5210x; 1.0600x over previous
import numpy as np
import jax
import jax.numpy as jnp
from jax.experimental import pallas as pl
from jax.experimental.pallas import tpu as pltpu

NQ = 8
DEPTH = 3
PI = 3.14159
BB = 256


def _dot(a, b):
    return jnp.dot(a, b, preferred_element_type=jnp.float32)


def _dott(a, b):
    return jax.lax.dot_general(a, b, (((1,), (1,)), ((), ())),
                               preferred_element_type=jnp.float32)


def _qcnn_body(xp, w2, w3, gf, ga, bp, th, qm, zpm,
               q2c, q2cb, bq, bqb, bf, bfb, iw, inb, f2, f2b, o):
    relu = jax.nn.relu
    xs = xp[...]
    lhs = jnp.concatenate([xs[0:28], xs[1:29], xs[2:30]],
                          axis=2).reshape(28 * BB, 252)

    def conv1(c):
        y4 = _dot(lhs, w2[c]).reshape(14, 2, BB, 448)
        return jnp.maximum(y4[:, 0], y4[:, 1])

    p1 = jnp.maximum(conv1(0), conv1(1))
    p1 = relu(p1 + bp[:, :448]).reshape(7, 2, BB, 448)
    pe = p1[:, 0]
    po = p1[:, 1]

    a_slabs = {
        (0, 0): pe[0:6], (0, 1): po[0:6], (0, 2): pe[1:7],
        (1, 0): po[0:6], (1, 1): pe[1:7], (1, 2): po[1:7],
    }
    m2 = None
    for rp in (0, 1):
        a_flat = [a_slabs[(rp, kh)].reshape(6 * BB, 448) for kh in range(3)]
        z = (_dot(a_flat[0], w3[0]) + _dot(a_flat[1], w3[1])
             + _dot(a_flat[2], w3[2]))
        zz = jnp.maximum(z[:, :384], z[:, 384:])
        m2 = zz if m2 is None else jnp.maximum(m2, zz)
    p2 = relu(m2 + bp[:, 448:832]).reshape(6, BB, 384)

    acc = _dot(p2[0], gf[0])
    aux = _dot(p2[0], ga[0])
    for t in range(1, 6):
        acc = acc + _dot(p2[t], gf[t])
        aux = aux + _dot(p2[t], ga[t])
    classical = relu(acc + bp[:, 832:1344])
    fractal = jnp.sin(aux[:, :64] * PI)
    rep = aux[:, 64:72]
    nrm = jnp.sqrt(jnp.sum(rep * rep, axis=1, keepdims=True))
    qin = rep / (nrm + 1e-8)

    col = jax.lax.broadcasted_iota(jnp.int32, (BB, 256), 1)
    state = (col == 0).astype(jnp.float32)
    cq = jnp.cos(0.5 * qin)
    sq = jnp.sin(0.5 * qin)
    thv = th[...]
    ct = jnp.cos(0.5 * thv)
    st = jnp.sin(0.5 * thv)
    for d in range(DEPTH):
        cth = ct[d:d + 1, :] * cq - st[d:d + 1, :] * sq
        sth = st[d:d + 1, :] * cq + ct[d:d + 1, :] * sq
        for j in range(4):
            qa, qb = 2 * j, 2 * j + 1
            ca, sa = cth[:, qa:qa + 1], sth[:, qa:qa + 1]
            cb, sb = cth[:, qb:qb + 1], sth[:, qb:qb + 1]
            if j < 3:
                s0 = state
                sA = _dot(state, qm[3 * j + 0])
                sB = _dot(state, qm[3 * j + 1])
                sC = _dot(state, qm[3 * j + 2])
            else:
                s0 = _dot(state, qm[9])
                sA = _dot(state, qm[10])
                sB = _dot(state, qm[11])
                sC = _dot(state, qm[12])
            state = ((ca * cb) * s0 + (sa * cb) * sA
                     + (ca * sb) * sB + (sa * sb) * sC)
    qout = _dot(state * state, zpm[...])

    qfeat = _dott(qout, q2c[...]) + q2cb[...]
    qf = jnp.tanh(_dott(qfeat, bq[...]) + bqb[...]) * \
        jnp.tanh(_dott(fractal, bf[...]) + bfb[...])
    iwv = iw[...]
    integrated = _dott(classical, iwv[:, :512]) + _dott(qf, iwv[:, 512:544]) \
        + inb[...]
    o[...] = _dott(integrated, f2[...]) + f2b[...]


def kernel(x, conv1_w, conv1_b, conv2_w, conv2_b, fc1_w, fc1_b, fc2_w, fc2_b,
           q2c_w, q2c_b, bq_w, bq_b, bf_w, bf_b, int_w, int_b, theta):
    f32 = jnp.float32
    B = x.shape[0]
    nb = B // BB

    xpad = jnp.pad(x, ((0, 0), (0, 0), (1, 1), (0, 0)))
    xp = xpad.transpose(2, 0, 1, 3).reshape(30, B, 84)

    kwv = np.arange(3)
    s2 = ((np.arange(28)[None, :, None, None] + 1 - kwv[:, None, None, None]
           - 2 * np.arange(14)[None, None, :, None]
           - np.arange(2)[None, None, None, :]) == 0).astype(np.float32)
    w2s = jnp.einsum('oihk,kpuc->chipou', conv1_w,
                     jnp.asarray(s2)).reshape(2, 252, 448)

    s3 = ((np.arange(14)[None, :, None, None] - kwv[:, None, None, None]
           - 2 * np.arange(6)[None, None, :, None]
           - np.arange(2)[None, None, None, :]) == 0).astype(np.float32)
    w3s = jnp.einsum('oihk,kpuc->hipcou', conv2_w,
                     jnp.asarray(s3)).reshape(3, 448, 768)

    gfc = fc1_w.reshape(512, 64, 6, 6).transpose(2, 1, 3, 0)
    gfc = gfc.reshape(6, 384, 512)
    gmean = np.kron(np.eye(64, dtype=np.float32),
                    np.ones((6, 1), dtype=np.float32)) / 36.0
    gmean = np.broadcast_to(gmean, (6, 384, 64))
    grep = np.zeros((6, 384, 8), dtype=np.float32)
    for k in range(8):
        grep[k // 6, (k % 6), k] = 1.0
    gaux = jnp.asarray(np.concatenate([gmean, grep], axis=2))

    idx = np.arange(256)
    pq = np.zeros((8, 256, 256), dtype=np.float32)
    for q in range(8):
        v = 1 << (7 - q)
        pq[q, idx ^ v, idx] = 1.0
    sg = np.where((idx[None, :] >> (7 - np.arange(8)[:, None])) & 1,
                  1.0, -1.0).astype(np.float32)
    t = idx.copy()
    for q in range(6, -1, -1):
        cv, tv = 1 << (7 - q), 1 << (6 - q)
        t = np.where(t & cv, t ^ tv, t)
    cmat = np.zeros((256, 256), dtype=np.float32)
    cmat[t, idx] = 1.0
    zpm = np.where((idx[:, None] >> (7 - np.arange(8)[None, :])) & 1,
                   -1.0, 1.0).astype(np.float32)
    qmat = np.zeros((13, 256, 256), dtype=np.float32)
    for j in range(4):
        a = pq[2 * j] * sg[2 * j][None, :]
        b = pq[2 * j + 1] * sg[2 * j + 1][None, :]
        c = a @ b
        if j < 3:
            qmat[3 * j + 0], qmat[3 * j + 1], qmat[3 * j + 2] = a, b, c
        else:
            qmat[9] = cmat
            qmat[10] = a @ cmat
            qmat[11] = b @ cmat
            qmat[12] = c @ cmat

    bias_pack = jnp.concatenate([jnp.repeat(conv1_b, 14),
                                 jnp.repeat(conv2_b, 6), fc1_b])[None]

    const2 = lambda i: (0, 0)
    const3 = lambda i: (0, 0, 0)
    in_specs = [
        pl.BlockSpec((30, BB, 84), lambda i: (0, i, 0)),
        pl.BlockSpec((2, 252, 448), const3),
        pl.BlockSpec((3, 448, 768), const3),
        pl.BlockSpec((6, 384, 512), const3),
        pl.BlockSpec((6, 384, 72), const3),
        pl.BlockSpec((1, 1344), const2),
        pl.BlockSpec((3, 8), const2),
        pl.BlockSpec((13, 256, 256), const3),
        pl.BlockSpec((256, 8), const2),
        pl.BlockSpec((64, 8), const2),
        pl.BlockSpec((1, 64), const2),
        pl.BlockSpec((32, 64), const2),
        pl.BlockSpec((1, 32), const2),
        pl.BlockSpec((32, 64), const2),
        pl.BlockSpec((1, 32), const2),
        pl.BlockSpec((512, 544), const2),
        pl.BlockSpec((1, 512), const2),
        pl.BlockSpec((10, 512), const2),
        pl.BlockSpec((1, 10), const2),
    ]
    out = pl.pallas_call(
        _qcnn_body,
        grid=(nb,),
        in_specs=in_specs,
        out_specs=pl.BlockSpec((BB, 10), lambda i: (i, 0)),
        out_shape=jax.ShapeDtypeStruct((B, 10), f32),
        compiler_params=pltpu.CompilerParams(
            dimension_semantics=("parallel",),
            allow_input_fusion=(True,) * 19,
            vmem_limit_bytes=56 * 1024 * 1024,
        ),
        name="qcnn_fused",
    )(xp, w2s, w3s, gfc, gaux, bias_pack, theta, jnp.asarray(qmat),
      jnp.asarray(zpm), q2c_w, q2c_b[None], bq_w, bq_b[None],
      bf_w, bf_b[None], int_w, int_b[None], fc2_w, fc2_b[None])
    return out
